# baseline (speedup 1.0000x reference)
import jax
import jax.numpy as jnp
from jax import lax
from jax.experimental import pallas as pl
from jax.experimental.pallas import tpu as pltpu
from jax.experimental.pallas import tpu_sc as plsc

NC, NS = 2, 16

N = 10000
E = 160000
D = 256
HD = D // 2

K = 125
NCHUNK = E // K
CPT_AGG = NCHUNK // NS
CPT_DEG = NCHUNK // (NC * NS)
NPAD = 10240
RPT = NPAD // NS

_mesh = plsc.VectorSubcoreMesh(
    core_axis_name="c", subcore_axis_name="s", num_cores=NC, num_subcores=NS
)


def _deg_body(dst_hbm, zeros_hbm, ones_hbm, deg_out, idx_v, ones_v, acc):
    c = lax.axis_index("c")
    s = lax.axis_index("s")
    t = c * NS + s
    pltpu.sync_copy(zeros_hbm.at[pl.ds(s * 640, 640)], acc.at[pl.ds(s * 640, 640)])
    pltpu.sync_copy(ones_hbm, ones_v)
    pltpu.sync_copy(dst_hbm.at[pl.ds(t * CPT_DEG, CPT_DEG)], idx_v)
    plsc.subcore_barrier()

    def body(j, carry):
        pltpu.sync_copy(ones_v, acc.at[idx_v.at[j]], add=True)
        return carry

    lax.fori_loop(0, CPT_DEG, body, 0)
    plsc.subcore_barrier()
    pltpu.sync_copy(acc.at[pl.ds(s * 640, 640)], deg_out.at[c, pl.ds(s * 640, 640)])


_deg_call = pl.kernel(
    _deg_body,
    out_type=jax.ShapeDtypeStruct((NC, NPAD), jnp.float32),
    mesh=_mesh,
    scratch_types=[
        pltpu.VMEM((CPT_DEG, K), jnp.int32),
        pltpu.VMEM((K,), jnp.float32),
        pltpu.VMEM_SHARED((NPAD,), jnp.float32),
    ],
)


QD = D // 4


def _agg_body(g4, src_hbm, dst_hbm, s_out, srcv, dstv, rows, acc, sem):
    c = lax.axis_index("c")
    s = lax.axis_index("s")
    rbase = s * RPT

    pltpu.sync_copy(src_hbm.at[pl.ds(s * CPT_AGG, CPT_AGG)], srcv)
    pltpu.sync_copy(dst_hbm.at[pl.ds(s * CPT_AGG, CPT_AGG)], dstv)

    def one_pass(p2, carry):
        g = g4.at[c, p2]
        pltpu.sync_copy(g4.at[c, p2, pl.ds(rbase, RPT)], acc.at[pl.ds(rbase, RPT)])
        plsc.subcore_barrier()
        pltpu.async_copy(g.at[srcv.at[0]], rows.at[0], sem)

        def body(j, carry2):
            p = j % 2
            pltpu.make_async_copy(g.at[srcv.at[j]], rows.at[p], sem).wait()

            @pl.when(j < CPT_AGG - 1)
            def _():
                pltpu.async_copy(g.at[srcv.at[j + 1]], rows.at[1 - p], sem)

            pltpu.sync_copy(rows.at[p], acc.at[dstv.at[j]], add=True)
            return carry2

        lax.fori_loop(0, CPT_AGG, body, 0)
        plsc.subcore_barrier()
        pltpu.sync_copy(acc.at[pl.ds(rbase, RPT)], s_out.at[c, p2, pl.ds(rbase, RPT)])
        plsc.subcore_barrier()
        return carry

    lax.fori_loop(0, 2, one_pass, 0)


_agg_call = pl.kernel(
    _agg_body,
    out_type=jax.ShapeDtypeStruct((NC, 2, NPAD, QD), jnp.float32),
    mesh=_mesh,
    scratch_types=[
        pltpu.VMEM((CPT_AGG, K), jnp.int32),
        pltpu.VMEM((CPT_AGG, K), jnp.int32),
        pltpu.VMEM((2, K, QD), jnp.float32),
        pltpu.VMEM_SHARED((NPAD, QD), jnp.float32),
        pltpu.SemaphoreType.DMA,
    ],
    compiler_params=pltpu.CompilerParams(use_tc_tiling_on_sc=False),
)


R = 1024
RO = 1000


def _dis_block(deg_ref, r):
    sl = pl.ds(r * R, R)
    return lax.rsqrt(deg_ref[0, sl] + deg_ref[1, sl] + 1.0)


def _mm1_body(deg_ref, x_ref, w_ref, g_ref, dis_ref):
    di = _dis_block(deg_ref, pl.program_id(0))
    dis_ref[...] = di[:, None]
    h = jnp.dot(x_ref[...], w_ref[...], preferred_element_type=jnp.float32)
    h = h * di[:, None]
    g_ref[0, 0] = h[:, 0 * QD : 1 * QD]
    g_ref[0, 1] = h[:, 1 * QD : 2 * QD]
    g_ref[1, 0] = h[:, 2 * QD : 3 * QD]
    g_ref[1, 1] = h[:, 3 * QD : 4 * QD]


_mm1_call = pl.pallas_call(
    _mm1_body,
    grid=(NPAD // R,),
    in_specs=[
        pl.BlockSpec((NC, NPAD), lambda r: (0, 0)),
        pl.BlockSpec((R, D), lambda r: (r, 0)),
        pl.BlockSpec((D, D), lambda r: (0, 0)),
    ],
    out_specs=[
        pl.BlockSpec((NC, 2, R, QD), lambda r: (0, 0, r, 0)),
        pl.BlockSpec((R, 1), lambda r: (r, 0)),
    ],
    out_shape=[
        jax.ShapeDtypeStruct((NC, 2, NPAD, QD), jnp.float32),
        jax.ShapeDtypeStruct((NPAD, 1), jnp.float32),
    ],
)


def _mm2_body(dis_ref, s_ref, b_ref, w_ref, g_ref):
    di = dis_ref[...]
    scat = jnp.concatenate(
        [s_ref[0, 0], s_ref[0, 1], s_ref[1, 0], s_ref[1, 1]], axis=1
    )
    h1 = jnp.maximum(scat * di + b_ref[...], 0.0)
    g = jnp.dot(h1, w_ref[...], preferred_element_type=jnp.float32) * di
    g_ref[0, 0] = g[:, 0 * QD : 1 * QD]
    g_ref[0, 1] = g[:, 1 * QD : 2 * QD]
    g_ref[1, 0] = g[:, 2 * QD : 3 * QD]
    g_ref[1, 1] = g[:, 3 * QD : 4 * QD]


_mm2_call = pl.pallas_call(
    _mm2_body,
    grid=(NPAD // R,),
    in_specs=[
        pl.BlockSpec((R, 1), lambda r: (r, 0)),
        pl.BlockSpec((NC, 2, R, QD), lambda r: (0, 0, r, 0)),
        pl.BlockSpec((1, D), lambda r: (0, 0)),
        pl.BlockSpec((D, D), lambda r: (0, 0)),
    ],
    out_specs=pl.BlockSpec((NC, 2, R, QD), lambda r: (0, 0, r, 0)),
    out_shape=jax.ShapeDtypeStruct((NC, 2, NPAD, QD), jnp.float32),
)


def _out_body(dis_ref, s_ref, b_ref, o_ref):
    scat = jnp.concatenate(
        [s_ref[0, 0], s_ref[0, 1], s_ref[1, 0], s_ref[1, 1]], axis=1
    )
    o_ref[...] = scat * dis_ref[...] + b_ref[...]


_out_call = pl.pallas_call(
    _out_body,
    grid=(N // RO,),
    in_specs=[
        pl.BlockSpec((RO, 1), lambda r: (r, 0)),
        pl.BlockSpec((NC, 2, RO, QD), lambda r: (0, 0, r, 0)),
        pl.BlockSpec((1, D), lambda r: (0, 0)),
    ],
    out_specs=pl.BlockSpec((RO, D), lambda r: (r, 0)),
    out_shape=jax.ShapeDtypeStruct((N, D), jnp.float32),
)


def kernel(x, edge_index, W1, b1, W2, b2):
    src = edge_index[0].reshape(NCHUNK, K)
    dst = edge_index[1].reshape(NCHUNK, K)
    zeros = jnp.zeros((NPAD,), jnp.float32)
    ones = jnp.ones((K,), jnp.float32)

    deg2 = _deg_call(dst, zeros, ones)
    g1, dis = _mm1_call(deg2, x, W1)
    s1 = _agg_call(g1, src, dst)
    g2 = _mm2_call(dis, s1, b1.reshape(1, D), W2)
    s2 = _agg_call(g2, src, dst)
    return _out_call(dis, s2, b2.reshape(1, D))


# speedup vs baseline: 1.1808x; 1.1808x over previous
import jax
import jax.numpy as jnp
from jax import lax
from jax.experimental import pallas as pl
from jax.experimental.pallas import tpu as pltpu
from jax.experimental.pallas import tpu_sc as plsc

NC, NS = 2, 16

N = 10000
E = 160000
D = 256
HD = D // 2

K = 125
NCHUNK = E // K
CPT_AGG = NCHUNK // NS
CPT_DEG = NCHUNK // (NC * NS)
NPAD = 10240
RPT = NPAD // NS

_mesh = plsc.VectorSubcoreMesh(
    core_axis_name="c", subcore_axis_name="s", num_cores=NC, num_subcores=NS
)


def _deg_body(dst_hbm, zeros_hbm, ones_hbm, deg_out, idx_v, ones_v, acc):
    c = lax.axis_index("c")
    s = lax.axis_index("s")
    t = c * NS + s
    pltpu.sync_copy(zeros_hbm.at[pl.ds(s * 640, 640)], acc.at[pl.ds(s * 640, 640)])
    pltpu.sync_copy(ones_hbm, ones_v)
    pltpu.sync_copy(dst_hbm.at[pl.ds(t * CPT_DEG, CPT_DEG)], idx_v)
    plsc.subcore_barrier()

    def body(j, carry):
        pltpu.sync_copy(ones_v, acc.at[idx_v.at[j]], add=True)
        return carry

    lax.fori_loop(0, CPT_DEG, body, 0)
    plsc.subcore_barrier()
    pltpu.sync_copy(acc.at[pl.ds(s * 640, 640)], deg_out.at[c, pl.ds(s * 640, 640)])


_deg_call = pl.kernel(
    _deg_body,
    out_type=jax.ShapeDtypeStruct((NC, NPAD), jnp.float32),
    mesh=_mesh,
    scratch_types=[
        pltpu.VMEM((CPT_DEG, K), jnp.int32),
        pltpu.VMEM((K,), jnp.float32),
        pltpu.VMEM_SHARED((NPAD,), jnp.float32),
    ],
)


def _agg_body(g3, src_hbm, dst_hbm, s_out, srcv, dstv, rows0, acc):
    c = lax.axis_index("c")
    s = lax.axis_index("s")
    g = g3.at[c]
    rbase = s * RPT

    pltpu.sync_copy(g3.at[c, pl.ds(rbase, RPT)], acc.at[pl.ds(rbase, RPT)])
    pltpu.sync_copy(src_hbm.at[pl.ds(s * CPT_AGG, CPT_AGG)], srcv)
    pltpu.sync_copy(dst_hbm.at[pl.ds(s * CPT_AGG, CPT_AGG)], dstv)
    plsc.subcore_barrier()

    def body(j, carry):
        pltpu.sync_copy(g.at[srcv.at[j]], rows0)
        pltpu.sync_copy(rows0, acc.at[dstv.at[j]], add=True)
        return carry

    lax.fori_loop(0, CPT_AGG, body, 0)
    plsc.subcore_barrier()
    pltpu.sync_copy(acc.at[pl.ds(rbase, RPT)], s_out.at[c, pl.ds(rbase, RPT)])


_agg_call = pl.kernel(
    _agg_body,
    out_type=jax.ShapeDtypeStruct((NC, NPAD, HD), jnp.float32),
    mesh=_mesh,
    scratch_types=[
        pltpu.VMEM((CPT_AGG, K), jnp.int32),
        pltpu.VMEM((CPT_AGG, K), jnp.int32),
        pltpu.VMEM((K, HD), jnp.float32),
        pltpu.VMEM_SHARED((NPAD, HD), jnp.float32),
    ],
)


R = 1024
RO = 1000


def _dis_block(deg_ref, r):
    sl = pl.ds(r * R, R)
    return lax.rsqrt(deg_ref[0, sl] + deg_ref[1, sl] + 1.0)


def _mm1_body(deg_ref, x_ref, w_ref, g_ref, dis_ref):
    di = _dis_block(deg_ref, pl.program_id(0))
    dis_ref[...] = di[:, None]
    h = jnp.dot(x_ref[...], w_ref[...], preferred_element_type=jnp.float32)
    h = h * di[:, None]
    g_ref[0] = h[:, :HD]
    g_ref[1] = h[:, HD:]


_mm1_call = pl.pallas_call(
    _mm1_body,
    grid=(NPAD // R,),
    in_specs=[
        pl.BlockSpec((NC, NPAD), lambda r: (0, 0)),
        pl.BlockSpec((R, D), lambda r: (r, 0)),
        pl.BlockSpec((D, D), lambda r: (0, 0)),
    ],
    out_specs=[
        pl.BlockSpec((NC, R, HD), lambda r: (0, r, 0)),
        pl.BlockSpec((R, 1), lambda r: (r, 0)),
    ],
    out_shape=[
        jax.ShapeDtypeStruct((NC, NPAD, HD), jnp.float32),
        jax.ShapeDtypeStruct((NPAD, 1), jnp.float32),
    ],
)


def _mm2_body(dis_ref, s_ref, b_ref, w_ref, g_ref):
    di = dis_ref[...]
    scat = jnp.concatenate([s_ref[0], s_ref[1]], axis=1)
    h1 = jnp.maximum(scat * di + b_ref[...], 0.0)
    g = jnp.dot(h1, w_ref[...], preferred_element_type=jnp.float32) * di
    g_ref[0] = g[:, :HD]
    g_ref[1] = g[:, HD:]


_mm2_call = pl.pallas_call(
    _mm2_body,
    grid=(NPAD // R,),
    in_specs=[
        pl.BlockSpec((R, 1), lambda r: (r, 0)),
        pl.BlockSpec((NC, R, HD), lambda r: (0, r, 0)),
        pl.BlockSpec((1, D), lambda r: (0, 0)),
        pl.BlockSpec((D, D), lambda r: (0, 0)),
    ],
    out_specs=pl.BlockSpec((NC, R, HD), lambda r: (0, r, 0)),
    out_shape=jax.ShapeDtypeStruct((NC, NPAD, HD), jnp.float32),
)


def _out_body(dis_ref, s_ref, b_ref, o_ref):
    scat = jnp.concatenate([s_ref[0], s_ref[1]], axis=1)
    o_ref[...] = scat * dis_ref[...] + b_ref[...]


_out_call = pl.pallas_call(
    _out_body,
    grid=(N // RO,),
    in_specs=[
        pl.BlockSpec((RO, 1), lambda r: (r, 0)),
        pl.BlockSpec((NC, RO, HD), lambda r: (0, r, 0)),
        pl.BlockSpec((1, D), lambda r: (0, 0)),
    ],
    out_specs=pl.BlockSpec((RO, D), lambda r: (r, 0)),
    out_shape=jax.ShapeDtypeStruct((N, D), jnp.float32),
)


def kernel(x, edge_index, W1, b1, W2, b2):
    src = edge_index[0].reshape(NCHUNK, K)
    dst = edge_index[1].reshape(NCHUNK, K)
    zeros = jnp.zeros((NPAD,), jnp.float32)
    ones = jnp.ones((K,), jnp.float32)

    deg2 = _deg_call(dst, zeros, ones)
    g1, dis = _mm1_call(deg2, x, W1)
    s1 = _agg_call(g1, src, dst)
    g2 = _mm2_call(dis, s1, b1.reshape(1, D), W2)
    s2 = _agg_call(g2, src, dst)
    return _out_call(dis, s2, b2.reshape(1, D))


# speedup vs baseline: 1.1985x; 1.0150x over previous
import jax
import jax.numpy as jnp
from jax import lax
from jax.experimental import pallas as pl
from jax.experimental.pallas import tpu as pltpu
from jax.experimental.pallas import tpu_sc as plsc

NC, NS = 2, 16

N = 10000
E = 160000
D = 256
HD = D // 2

K = 125
NCHUNK = E // K
CPT_AGG = NCHUNK // NS
CPT_DEG = NCHUNK // (NC * NS)
NPAD = 10240
RPT = NPAD // NS

_mesh = plsc.VectorSubcoreMesh(
    core_axis_name="c", subcore_axis_name="s", num_cores=NC, num_subcores=NS
)


def _deg_body(dst_hbm, zeros_hbm, ones_hbm, deg_out, idx_v, ones_v, acc):
    c = lax.axis_index("c")
    s = lax.axis_index("s")
    t = c * NS + s
    pltpu.sync_copy(zeros_hbm.at[pl.ds(s * 640, 640)], acc.at[pl.ds(s * 640, 640)])
    pltpu.sync_copy(ones_hbm, ones_v)
    pltpu.sync_copy(dst_hbm.at[pl.ds(t * CPT_DEG, CPT_DEG)], idx_v)
    plsc.subcore_barrier()

    def body(j, carry):
        pltpu.sync_copy(ones_v, acc.at[idx_v.at[j]], add=True)
        return carry

    lax.fori_loop(0, CPT_DEG, body, 0)
    plsc.subcore_barrier()
    pltpu.sync_copy(acc.at[pl.ds(s * 640, 640)], deg_out.at[c, pl.ds(s * 640, 640)])


_deg_call = pl.kernel(
    _deg_body,
    out_type=jax.ShapeDtypeStruct((NC, NPAD), jnp.float32),
    mesh=_mesh,
    scratch_types=[
        pltpu.VMEM((CPT_DEG, K), jnp.int32),
        pltpu.VMEM((K,), jnp.float32),
        pltpu.VMEM_SHARED((NPAD,), jnp.float32),
    ],
)


def _agg_body(g3, src_hbm, dst_hbm, s_out, srcv, dstv, rows0, acc):
    c = lax.axis_index("c")
    s = lax.axis_index("s")
    g = g3.at[c]
    rbase = s * RPT

    pltpu.sync_copy(g3.at[c, pl.ds(rbase, RPT)], acc.at[pl.ds(rbase, RPT)])
    pltpu.sync_copy(src_hbm.at[pl.ds(s * CPT_AGG, CPT_AGG)], srcv)
    pltpu.sync_copy(dst_hbm.at[pl.ds(s * CPT_AGG, CPT_AGG)], dstv)
    plsc.subcore_barrier()

    def body(j, carry):
        pltpu.sync_copy(g.at[srcv.at[j]], rows0)
        pltpu.sync_copy(rows0, acc.at[dstv.at[j]], add=True)
        return carry

    lax.fori_loop(0, CPT_AGG, body, 0)
    plsc.subcore_barrier()
    pltpu.sync_copy(acc.at[pl.ds(rbase, RPT)], s_out.at[c, pl.ds(rbase, RPT)])


_agg_call = pl.kernel(
    _agg_body,
    out_type=jax.ShapeDtypeStruct((NC, NPAD, HD), jnp.float32),
    mesh=_mesh,
    scratch_types=[
        pltpu.VMEM((CPT_AGG, K), jnp.int32),
        pltpu.VMEM((CPT_AGG, K), jnp.int32),
        pltpu.VMEM((K, HD), jnp.float32),
        pltpu.VMEM_SHARED((NPAD, HD), jnp.float32),
    ],
)


R = 2048
RO = 2000


def _dis_block(deg_ref, r):
    sl = pl.ds(r * R, R)
    return lax.rsqrt(deg_ref[0, sl] + deg_ref[1, sl] + 1.0)


def _mm1_body(deg_ref, x_ref, w_ref, g_ref, dis_ref):
    di = _dis_block(deg_ref, pl.program_id(0))
    dis_ref[...] = di[:, None]
    h = jnp.dot(x_ref[...], w_ref[...], preferred_element_type=jnp.float32)
    h = h * di[:, None]
    g_ref[0] = h[:, :HD]
    g_ref[1] = h[:, HD:]


_mm1_call = pl.pallas_call(
    _mm1_body,
    grid=(NPAD // R,),
    in_specs=[
        pl.BlockSpec((NC, NPAD), lambda r: (0, 0)),
        pl.BlockSpec((R, D), lambda r: (r, 0)),
        pl.BlockSpec((D, D), lambda r: (0, 0)),
    ],
    out_specs=[
        pl.BlockSpec((NC, R, HD), lambda r: (0, r, 0)),
        pl.BlockSpec((R, 1), lambda r: (r, 0)),
    ],
    out_shape=[
        jax.ShapeDtypeStruct((NC, NPAD, HD), jnp.float32),
        jax.ShapeDtypeStruct((NPAD, 1), jnp.float32),
    ],
)


def _mm2_body(dis_ref, s_ref, b_ref, w_ref, g_ref):
    di = dis_ref[...]
    scat = jnp.concatenate([s_ref[0], s_ref[1]], axis=1)
    h1 = jnp.maximum(scat * di + b_ref[...], 0.0)
    g = jnp.dot(h1, w_ref[...], preferred_element_type=jnp.float32) * di
    g_ref[0] = g[:, :HD]
    g_ref[1] = g[:, HD:]


_mm2_call = pl.pallas_call(
    _mm2_body,
    grid=(NPAD // R,),
    in_specs=[
        pl.BlockSpec((R, 1), lambda r: (r, 0)),
        pl.BlockSpec((NC, R, HD), lambda r: (0, r, 0)),
        pl.BlockSpec((1, D), lambda r: (0, 0)),
        pl.BlockSpec((D, D), lambda r: (0, 0)),
    ],
    out_specs=pl.BlockSpec((NC, R, HD), lambda r: (0, r, 0)),
    out_shape=jax.ShapeDtypeStruct((NC, NPAD, HD), jnp.float32),
)


def _out_body(dis_ref, s_ref, b_ref, o_ref):
    scat = jnp.concatenate([s_ref[0], s_ref[1]], axis=1)
    o_ref[...] = scat * dis_ref[...] + b_ref[...]


_out_call = pl.pallas_call(
    _out_body,
    grid=(N // RO,),
    in_specs=[
        pl.BlockSpec((RO, 1), lambda r: (r, 0)),
        pl.BlockSpec((NC, RO, HD), lambda r: (0, r, 0)),
        pl.BlockSpec((1, D), lambda r: (0, 0)),
    ],
    out_specs=pl.BlockSpec((RO, D), lambda r: (r, 0)),
    out_shape=jax.ShapeDtypeStruct((N, D), jnp.float32),
)


def kernel(x, edge_index, W1, b1, W2, b2):
    src = edge_index[0].reshape(NCHUNK, K)
    dst = edge_index[1].reshape(NCHUNK, K)
    zeros = jnp.zeros((NPAD,), jnp.float32)
    ones = jnp.ones((K,), jnp.float32)

    deg2 = _deg_call(dst, zeros, ones)
    g1, dis = _mm1_call(deg2, x, W1)
    s1 = _agg_call(g1, src, dst)
    g2 = _mm2_call(dis, s1, b1.reshape(1, D), W2)
    s2 = _agg_call(g2, src, dst)
    return _out_call(dis, s2, b2.reshape(1, D))


# speedup vs baseline: 1.2091x; 1.0088x over previous
import jax
import jax.numpy as jnp
from jax import lax
from jax.experimental import pallas as pl
from jax.experimental.pallas import tpu as pltpu
from jax.experimental.pallas import tpu_sc as plsc

NC, NS = 2, 16

N = 10000
E = 160000
D = 256
HD = D // 2

K = 125
NCHUNK = E // K
CPT_AGG = NCHUNK // NS
CPT_DEG = NCHUNK // (NC * NS)
NPAD = 10240
RPT = NPAD // NS

_mesh = plsc.VectorSubcoreMesh(
    core_axis_name="c", subcore_axis_name="s", num_cores=NC, num_subcores=NS
)


def _deg_body(dst_hbm, zeros_hbm, ones_hbm, deg_out, idx_v, ones_v, acc):
    c = lax.axis_index("c")
    s = lax.axis_index("s")
    t = c * NS + s
    pltpu.sync_copy(zeros_hbm.at[pl.ds(s * 640, 640)], acc.at[pl.ds(s * 640, 640)])
    pltpu.sync_copy(ones_hbm, ones_v)
    pltpu.sync_copy(dst_hbm.at[pl.ds(t * CPT_DEG, CPT_DEG)], idx_v)
    plsc.subcore_barrier()

    def body(j, carry):
        pltpu.sync_copy(ones_v, acc.at[idx_v.at[j]], add=True)
        return carry

    lax.fori_loop(0, CPT_DEG, body, 0)
    plsc.subcore_barrier()
    pltpu.sync_copy(acc.at[pl.ds(s * 640, 640)], deg_out.at[c, pl.ds(s * 640, 640)])


_deg_call = pl.kernel(
    _deg_body,
    out_type=jax.ShapeDtypeStruct((NC, NPAD), jnp.float32),
    mesh=_mesh,
    scratch_types=[
        pltpu.VMEM((CPT_DEG, K), jnp.int32),
        pltpu.VMEM((K,), jnp.float32),
        pltpu.VMEM_SHARED((NPAD,), jnp.float32),
    ],
)


def _agg_body(g3, src_hbm, dst_hbm, s_out, srcv, dstv, rows0, acc):
    c = lax.axis_index("c")
    s = lax.axis_index("s")
    g = g3.at[c]
    rbase = s * RPT

    pltpu.sync_copy(g3.at[c, pl.ds(rbase, RPT)], acc.at[pl.ds(rbase, RPT)])
    pltpu.sync_copy(src_hbm.at[pl.ds(s * CPT_AGG, CPT_AGG)], srcv)
    pltpu.sync_copy(dst_hbm.at[pl.ds(s * CPT_AGG, CPT_AGG)], dstv)
    plsc.subcore_barrier()

    def body(j, carry):
        pltpu.sync_copy(g.at[srcv.at[j]], rows0)
        pltpu.sync_copy(rows0, acc.at[dstv.at[j]], add=True)
        return carry

    lax.fori_loop(0, CPT_AGG, body, 0)
    plsc.subcore_barrier()
    pltpu.sync_copy(acc.at[pl.ds(rbase, RPT)], s_out.at[c, pl.ds(rbase, RPT)])


_agg_call = pl.kernel(
    _agg_body,
    out_type=jax.ShapeDtypeStruct((NC, NPAD, HD), jnp.float32),
    mesh=_mesh,
    scratch_types=[
        pltpu.VMEM((CPT_AGG, K), jnp.int32),
        pltpu.VMEM((CPT_AGG, K), jnp.int32),
        pltpu.VMEM((K, HD), jnp.float32),
        pltpu.VMEM_SHARED((NPAD, HD), jnp.float32),
    ],
)


R = 2560
RO = 5000


def _dis_block(deg_ref, r):
    sl = pl.ds(r * R, R)
    return lax.rsqrt(deg_ref[0, sl] + deg_ref[1, sl] + 1.0)


def _mm1_body(deg_ref, x_ref, w_ref, g_ref, dis_ref):
    di = _dis_block(deg_ref, pl.program_id(0))
    dis_ref[...] = di[:, None]
    h = jnp.dot(x_ref[...], w_ref[...], preferred_element_type=jnp.float32)
    h = h * di[:, None]
    g_ref[0] = h[:, :HD]
    g_ref[1] = h[:, HD:]


_mm1_call = pl.pallas_call(
    _mm1_body,
    grid=(NPAD // R,),
    in_specs=[
        pl.BlockSpec((NC, NPAD), lambda r: (0, 0)),
        pl.BlockSpec((R, D), lambda r: (r, 0)),
        pl.BlockSpec((D, D), lambda r: (0, 0)),
    ],
    out_specs=[
        pl.BlockSpec((NC, R, HD), lambda r: (0, r, 0)),
        pl.BlockSpec((R, 1), lambda r: (r, 0)),
    ],
    out_shape=[
        jax.ShapeDtypeStruct((NC, NPAD, HD), jnp.float32),
        jax.ShapeDtypeStruct((NPAD, 1), jnp.float32),
    ],
)


def _mm2_body(dis_ref, s_ref, b_ref, w_ref, g_ref):
    di = dis_ref[...]
    scat = jnp.concatenate([s_ref[0], s_ref[1]], axis=1)
    h1 = jnp.maximum(scat * di + b_ref[...], 0.0)
    g = jnp.dot(h1, w_ref[...], preferred_element_type=jnp.float32) * di
    g_ref[0] = g[:, :HD]
    g_ref[1] = g[:, HD:]


_mm2_call = pl.pallas_call(
    _mm2_body,
    grid=(NPAD // R,),
    in_specs=[
        pl.BlockSpec((R, 1), lambda r: (r, 0)),
        pl.BlockSpec((NC, R, HD), lambda r: (0, r, 0)),
        pl.BlockSpec((1, D), lambda r: (0, 0)),
        pl.BlockSpec((D, D), lambda r: (0, 0)),
    ],
    out_specs=pl.BlockSpec((NC, R, HD), lambda r: (0, r, 0)),
    out_shape=jax.ShapeDtypeStruct((NC, NPAD, HD), jnp.float32),
)


def _out_body(dis_ref, s_ref, b_ref, o_ref):
    scat = jnp.concatenate([s_ref[0], s_ref[1]], axis=1)
    o_ref[...] = scat * dis_ref[...] + b_ref[...]


_out_call = pl.pallas_call(
    _out_body,
    grid=(N // RO,),
    in_specs=[
        pl.BlockSpec((RO, 1), lambda r: (r, 0)),
        pl.BlockSpec((NC, RO, HD), lambda r: (0, r, 0)),
        pl.BlockSpec((1, D), lambda r: (0, 0)),
    ],
    out_specs=pl.BlockSpec((RO, D), lambda r: (r, 0)),
    out_shape=jax.ShapeDtypeStruct((N, D), jnp.float32),
)


def kernel(x, edge_index, W1, b1, W2, b2):
    src = edge_index[0].reshape(NCHUNK, K)
    dst = edge_index[1].reshape(NCHUNK, K)
    zeros = jnp.zeros((NPAD,), jnp.float32)
    ones = jnp.ones((K,), jnp.float32)

    deg2 = _deg_call(dst, zeros, ones)
    g1, dis = _mm1_call(deg2, x, W1)
    s1 = _agg_call(g1, src, dst)
    g2 = _mm2_call(dis, s1, b1.reshape(1, D), W2)
    s2 = _agg_call(g2, src, dst)
    return _out_call(dis, s2, b2.reshape(1, D))


# speedup vs baseline: 1.2158x; 1.0056x over previous
import jax
import jax.numpy as jnp
from jax import lax
from jax.experimental import pallas as pl
from jax.experimental.pallas import tpu as pltpu
from jax.experimental.pallas import tpu_sc as plsc

NC, NS = 2, 16

N = 10000
E = 160000
D = 256
HD = D // 2

K = 125
NCHUNK = E // K
CPT_AGG = NCHUNK // NS
CPT_DEG = NCHUNK // (NC * NS)
NPAD = 10240
RPT = NPAD // NS

_mesh = plsc.VectorSubcoreMesh(
    core_axis_name="c", subcore_axis_name="s", num_cores=NC, num_subcores=NS
)


def _deg_body(dst_hbm, zeros_hbm, ones_hbm, deg_out, idx_v, ones_v, acc):
    c = lax.axis_index("c")
    s = lax.axis_index("s")
    t = c * NS + s
    pltpu.sync_copy(zeros_hbm.at[pl.ds(s * 640, 640)], acc.at[pl.ds(s * 640, 640)])
    pltpu.sync_copy(ones_hbm, ones_v)
    pltpu.sync_copy(dst_hbm.at[pl.ds(t * CPT_DEG, CPT_DEG)], idx_v)
    plsc.subcore_barrier()

    def body(j, carry):
        pltpu.sync_copy(ones_v, acc.at[idx_v.at[j]], add=True)
        return carry

    lax.fori_loop(0, CPT_DEG, body, 0)
    plsc.subcore_barrier()
    pltpu.sync_copy(acc.at[pl.ds(s * 640, 640)], deg_out.at[c, pl.ds(s * 640, 640)])


_deg_call = pl.kernel(
    _deg_body,
    out_type=jax.ShapeDtypeStruct((NC, NPAD), jnp.float32),
    mesh=_mesh,
    scratch_types=[
        pltpu.VMEM((CPT_DEG, K), jnp.int32),
        pltpu.VMEM((K,), jnp.float32),
        pltpu.VMEM_SHARED((NPAD,), jnp.float32),
    ],
)


def _agg_body(g3, src_hbm, dst_hbm, s_out, srcv, dstv, rows0, acc):
    c = lax.axis_index("c")
    s = lax.axis_index("s")
    g = g3.at[c]
    rbase = s * RPT

    pltpu.sync_copy(g3.at[c, pl.ds(rbase, RPT)], acc.at[pl.ds(rbase, RPT)])
    pltpu.sync_copy(src_hbm.at[pl.ds(s * CPT_AGG, CPT_AGG)], srcv)
    pltpu.sync_copy(dst_hbm.at[pl.ds(s * CPT_AGG, CPT_AGG)], dstv)
    plsc.subcore_barrier()

    def body(j, carry):
        pltpu.sync_copy(g.at[srcv.at[j]], rows0)
        pltpu.sync_copy(rows0, acc.at[dstv.at[j]], add=True)
        return carry

    lax.fori_loop(0, CPT_AGG, body, 0)
    plsc.subcore_barrier()
    pltpu.sync_copy(acc.at[pl.ds(rbase, RPT)], s_out.at[c, pl.ds(rbase, RPT)])


_agg_call = pl.kernel(
    _agg_body,
    out_type=jax.ShapeDtypeStruct((NC, NPAD, HD), jnp.float32),
    mesh=_mesh,
    scratch_types=[
        pltpu.VMEM((CPT_AGG, K), jnp.int32),
        pltpu.VMEM((CPT_AGG, K), jnp.int32),
        pltpu.VMEM((K, HD), jnp.float32),
        pltpu.VMEM_SHARED((NPAD, HD), jnp.float32),
    ],
)


R = 5120
RO = 5000


def _dis_block(deg_ref, r):
    sl = pl.ds(r * R, R)
    return lax.rsqrt(deg_ref[0, sl] + deg_ref[1, sl] + 1.0)


def _mm1_body(deg_ref, x_ref, w_ref, g_ref, dis_ref):
    di = _dis_block(deg_ref, pl.program_id(0))
    dis_ref[...] = di[:, None]
    h = jnp.dot(x_ref[...], w_ref[...], preferred_element_type=jnp.float32)
    h = h * di[:, None]
    g_ref[0] = h[:, :HD]
    g_ref[1] = h[:, HD:]


_mm1_call = pl.pallas_call(
    _mm1_body,
    grid=(NPAD // R,),
    in_specs=[
        pl.BlockSpec((NC, NPAD), lambda r: (0, 0)),
        pl.BlockSpec((R, D), lambda r: (r, 0)),
        pl.BlockSpec((D, D), lambda r: (0, 0)),
    ],
    out_specs=[
        pl.BlockSpec((NC, R, HD), lambda r: (0, r, 0)),
        pl.BlockSpec((R, 1), lambda r: (r, 0)),
    ],
    out_shape=[
        jax.ShapeDtypeStruct((NC, NPAD, HD), jnp.float32),
        jax.ShapeDtypeStruct((NPAD, 1), jnp.float32),
    ],
)


def _mm2_body(dis_ref, s_ref, b_ref, w_ref, g_ref):
    di = dis_ref[...]
    scat = jnp.concatenate([s_ref[0], s_ref[1]], axis=1)
    h1 = jnp.maximum(scat * di + b_ref[...], 0.0)
    g = jnp.dot(h1, w_ref[...], preferred_element_type=jnp.float32) * di
    g_ref[0] = g[:, :HD]
    g_ref[1] = g[:, HD:]


_mm2_call = pl.pallas_call(
    _mm2_body,
    grid=(NPAD // R,),
    in_specs=[
        pl.BlockSpec((R, 1), lambda r: (r, 0)),
        pl.BlockSpec((NC, R, HD), lambda r: (0, r, 0)),
        pl.BlockSpec((1, D), lambda r: (0, 0)),
        pl.BlockSpec((D, D), lambda r: (0, 0)),
    ],
    out_specs=pl.BlockSpec((NC, R, HD), lambda r: (0, r, 0)),
    out_shape=jax.ShapeDtypeStruct((NC, NPAD, HD), jnp.float32),
)


def _out_body(dis_ref, s_ref, b_ref, o_ref):
    scat = jnp.concatenate([s_ref[0], s_ref[1]], axis=1)
    o_ref[...] = scat * dis_ref[...] + b_ref[...]


_out_call = pl.pallas_call(
    _out_body,
    grid=(N // RO,),
    in_specs=[
        pl.BlockSpec((RO, 1), lambda r: (r, 0)),
        pl.BlockSpec((NC, RO, HD), lambda r: (0, r, 0)),
        pl.BlockSpec((1, D), lambda r: (0, 0)),
    ],
    out_specs=pl.BlockSpec((RO, D), lambda r: (r, 0)),
    out_shape=jax.ShapeDtypeStruct((N, D), jnp.float32),
)


def kernel(x, edge_index, W1, b1, W2, b2):
    src = edge_index[0].reshape(NCHUNK, K)
    dst = edge_index[1].reshape(NCHUNK, K)
    zeros = jnp.zeros((NPAD,), jnp.float32)
    ones = jnp.ones((K,), jnp.float32)

    deg2 = _deg_call(dst, zeros, ones)
    g1, dis = _mm1_call(deg2, x, W1)
    s1 = _agg_call(g1, src, dst)
    g2 = _mm2_call(dis, s1, b1.reshape(1, D), W2)
    s2 = _agg_call(g2, src, dst)
    return _out_call(dis, s2, b2.reshape(1, D))
